# baseline (speedup 1.0000x reference)
import functools

import jax
import jax.numpy as jnp
from jax import lax
from jax.experimental import pallas as pl
from jax.experimental.pallas import tpu as pltpu
from jax.experimental.pallas import tpu_sc as plsc

D_S = 64
D_F = 300
WT = 432
FCOL = 2 * D_S
ACOL = 2 * D_S + D_F
C = 40
NGF = (D_F + 15) // 16


def _build(b, s):
    num_rows = b * s
    info = plsc.get_sparse_core_info()
    nc, ns, nl = info.num_cores, info.num_subcores, info.num_lanes
    nw = nc * ns
    assert num_rows % (nw * C) == 0 and s % C == 0
    cpw = num_rows // (nw * C)

    mesh = plsc.VectorSubcoreMesh(core_axis_name="c", subcore_axis_name="s")

    @functools.partial(
        pl.kernel,
        mesh=mesh,
        compiler_params=pltpu.CompilerParams(use_tc_tiling_on_sc=False,
                                             needs_layout_passes=False),
        out_type=[
            jax.ShapeDtypeStruct((b, s, D_S), jnp.float32),
            jax.ShapeDtypeStruct((b, s, D_S), jnp.float32),
            jax.ShapeDtypeStruct((b, s), jnp.float32),
            jax.ShapeDtypeStruct((b, s, D_F), jnp.float32),
        ],
        scratch_types=[
            pltpu.VMEM((1, cpw, C), jnp.int32),
            pltpu.VMEM((C, WT), jnp.float32),
            pltpu.VMEM((C, D_F), jnp.float32),
            pltpu.VMEM((C,), jnp.float32),
            pltpu.SemaphoreType.DMA,
        ],
    )
    def gather_kernel(idx_hbm, tab_hbm,
                      mu_o, lv_o, al_o, feat_o,
                      idx_v, buf_v, pk_v, al_v, sem):
        wid = lax.axis_index("s") * nc + lax.axis_index("c")
        crow = wid * cpw
        pltpu.sync_copy(idx_hbm.at[pl.ds(wid, 1)], idx_v)

        lane = lax.broadcasted_iota(jnp.int32, (nl,), 0)
        acols = jnp.full((nl,), ACOL, dtype=jnp.int32)
        tailmask = lane < (D_F - (NGF - 1) * nl)

        def chunk(j, carry):
            base = (crow + j) * C
            b0 = base // s
            s0 = base - b0 * s
            idx_row = idx_v.at[0, j]
            pltpu.async_copy(tab_hbm.at[idx_row], buf_v, sem).wait()

            for r0 in (0, nl, C - nl):
                v = plsc.load_gather(buf_v, [lane + r0, acols])
                al_v[pl.ds(r0, nl)] = 1.0 / (1.0 + jnp.exp(-v))

            def pack_row(r, carry2):
                rvec = jnp.full((nl,), r, dtype=jnp.int32)
                for k in range(NGF):
                    v = plsc.load_gather(buf_v, [rvec, lane + (FCOL + k * nl)])
                    if k == NGF - 1:
                        plsc.store_scatter(pk_v, [rvec, lane + k * nl], v,
                                           mask=tailmask)
                    else:
                        plsc.store_scatter(pk_v, [rvec, lane + k * nl], v)
                return carry2

            lax.fori_loop(0, C, pack_row, 0)

            pltpu.sync_copy(buf_v.at[:, pl.ds(0, D_S)],
                            mu_o.at[b0, pl.ds(s0, C)])
            pltpu.sync_copy(buf_v.at[:, pl.ds(D_S, D_S)],
                            lv_o.at[b0, pl.ds(s0, C)])
            pltpu.sync_copy(al_v, al_o.at[b0, pl.ds(s0, C)])
            pltpu.sync_copy(pk_v, feat_o.at[b0, pl.ds(s0, C)])
            return carry

        lax.fori_loop(0, cpw, chunk, 0)

    return gather_kernel


def kernel(indices, mu, log_var, raw_alpha, features):
    b, s = indices.shape
    n = b * s
    v = mu.shape[0]
    info = plsc.get_sparse_core_info()
    nw = info.num_cores * info.num_subcores
    idx = indices.astype(jnp.int32).reshape(nw, n // (nw * C), C)
    tab = jnp.concatenate(
        [mu, log_var, features, raw_alpha[:, None],
         jnp.zeros((v, WT - ACOL - 1), jnp.float32)], axis=1)
    gk = _build(b, s)
    return tuple(gk(idx, tab))


# speedup vs baseline: 1.1469x; 1.1469x over previous
import functools

import jax
import jax.numpy as jnp
from jax import lax
from jax.experimental import pallas as pl
from jax.experimental.pallas import tpu as pltpu
from jax.experimental.pallas import tpu_sc as plsc

D_S = 64
D_F = 300
WT = 512
FCOL = 2 * D_S
ACOL = 2 * D_S + D_F
C = 40
NGF = (D_F + 15) // 16


def _build(b, s):
    num_rows = b * s
    info = plsc.get_sparse_core_info()
    nc, ns, nl = info.num_cores, info.num_subcores, info.num_lanes
    nw = nc * ns
    cpw = num_rows // (nw * C)
    bpw = b // nw

    mesh = plsc.VectorSubcoreMesh(core_axis_name="c", subcore_axis_name="s")

    @functools.partial(
        pl.kernel,
        mesh=mesh,
        compiler_params=pltpu.CompilerParams(needs_layout_passes=False),
        out_type=[
            jax.ShapeDtypeStruct((b, s, D_S), jnp.float32),
            jax.ShapeDtypeStruct((b, s, D_S), jnp.float32),
            jax.ShapeDtypeStruct((b, s), jnp.float32),
            jax.ShapeDtypeStruct((b, s, D_F), jnp.float32),
        ],
        scratch_types=[
            pltpu.VMEM((1, cpw, C), jnp.int32),
            pltpu.VMEM((C, WT), jnp.float32),
            pltpu.VMEM((C, D_S), jnp.float32),
            pltpu.VMEM((C, D_S), jnp.float32),
            pltpu.VMEM((C, D_F), jnp.float32),
            pltpu.VMEM((bpw, s), jnp.float32),
            pltpu.SemaphoreType.DMA,
        ],
    )
    def gather_kernel(idx_hbm, tab_hbm,
                      mu_o, lv_o, al_o, feat_o,
                      idx_v, buf_v, mu_v, lv_v, pk_v, al_v, sem):
        wid = lax.axis_index("s") * nc + lax.axis_index("c")
        crow = wid * cpw
        b_lo = wid * bpw
        pltpu.sync_copy(idx_hbm.at[pl.ds(wid, 1)], idx_v)

        lane = lax.broadcasted_iota(jnp.int32, (nl,), 0)
        acols = jnp.full((nl,), ACOL, dtype=jnp.int32)
        tailmask = lane < (D_F - (NGF - 1) * nl)

        def chunk(j, carry):
            base = (crow + j) * C
            b0 = base // s
            s0 = base - b0 * s
            idx_row = idx_v.at[0, j]
            pltpu.async_copy(tab_hbm.at[idx_row], buf_v, sem).wait()

            for r0 in (0, nl, C - nl):
                v = plsc.load_gather(buf_v, [lane + r0, acols])
                sig = 1.0 / (1.0 + jnp.exp(-v))
                plsc.store_scatter(
                    al_v, [jnp.full((nl,), b0 - b_lo, jnp.int32),
                           lane + (s0 + r0)], sig)

            def pack_row(r, carry2):
                rvec = jnp.full((nl,), r, dtype=jnp.int32)
                for k in range(D_S // nl):
                    v = plsc.load_gather(buf_v, [rvec, lane + k * nl])
                    plsc.store_scatter(mu_v, [rvec, lane + k * nl], v)
                for k in range(D_S // nl):
                    v = plsc.load_gather(buf_v, [rvec, lane + (D_S + k * nl)])
                    plsc.store_scatter(lv_v, [rvec, lane + k * nl], v)
                for k in range(NGF):
                    v = plsc.load_gather(buf_v, [rvec, lane + (FCOL + k * nl)])
                    if k == NGF - 1:
                        plsc.store_scatter(pk_v, [rvec, lane + k * nl], v,
                                           mask=tailmask)
                    else:
                        plsc.store_scatter(pk_v, [rvec, lane + k * nl], v)
                return carry2

            lax.fori_loop(0, C, pack_row, 0)

            pltpu.sync_copy(mu_v, mu_o.at[b0, pl.ds(s0, C)])
            pltpu.sync_copy(lv_v, lv_o.at[b0, pl.ds(s0, C)])
            pltpu.sync_copy(pk_v, feat_o.at[b0, pl.ds(s0, C)])
            return carry

        lax.fori_loop(0, cpw, chunk, 0)
        pltpu.sync_copy(al_v, al_o.at[pl.ds(b_lo, bpw)])

    return gather_kernel


def kernel(indices, mu, log_var, raw_alpha, features):
    b, s = indices.shape
    n = b * s
    v = mu.shape[0]
    info = plsc.get_sparse_core_info()
    nw = info.num_cores * info.num_subcores
    idx = indices.astype(jnp.int32).reshape(nw, n // (nw * C), C)
    tab = jnp.concatenate(
        [mu, log_var, features, raw_alpha[:, None],
         jnp.zeros((v, WT - ACOL - 1), jnp.float32)], axis=1)
    gk = _build(b, s)
    return tuple(gk(idx, tab))
